# baseline (speedup 1.0000x reference)
import jax
import jax.numpy as jnp
from jax.experimental import pallas as pl
from jax.experimental.pallas import tpu as pltpu

_EPS = 1e-6
_TB = 256
_FB = 256


def _moe_body(xt_ref, rms_ref, rw_ref, rb_ref, g_ref, u_ref, d_ref,
              out_ref, comb_ref, gbf, ubf, dbf):
    e = pl.program_id(0)
    f = pl.program_id(1)
    t = pl.program_id(2)
    ts = t * _TB

    x = xt_ref[pl.ds(ts, _TB), :]
    var = jnp.mean(x * x, axis=1, keepdims=True)
    xn = (x * jax.lax.rsqrt(var + _EPS)) * rms_ref[:]

    @pl.when(jnp.logical_and(e == 0, f == 0))
    def _router():
        logits = jnp.dot(xn, rw_ref[:], preferred_element_type=jnp.float32)
        scores = jax.nn.sigmoid(logits)
        choice = scores + rb_ref[:]
        iota = jax.lax.broadcasted_iota(jnp.int32, (_TB, 128), 1)
        m1 = jnp.max(choice, axis=1, keepdims=True)
        i1 = jnp.min(jnp.where(choice == m1, iota, 128), axis=1, keepdims=True)
        mask1 = iota == i1
        choice2 = jnp.where(mask1, -jnp.inf, choice)
        m2 = jnp.max(choice2, axis=1, keepdims=True)
        i2 = jnp.min(jnp.where(choice2 == m2, iota, 128), axis=1, keepdims=True)
        mask2 = iota == i2
        a1 = jnp.sum(jnp.where(mask1, scores, 0.0), axis=1, keepdims=True)
        a2 = jnp.sum(jnp.where(mask2, scores, 0.0), axis=1, keepdims=True)
        den = a1 + a2 + 1e-9
        comb_ref[pl.ds(ts, _TB), :] = (jnp.where(mask1, a1 / den, 0.0)
                                       + jnp.where(mask2, a2 / den, 0.0))

    @pl.when(t == 0)
    def _cast_weights():
        gbf[:] = g_ref[0].astype(jnp.bfloat16)
        ubf[:] = u_ref[0].astype(jnp.bfloat16)
        dbf[:] = d_ref[0].astype(jnp.bfloat16)

    xnb = xn.astype(jnp.bfloat16)
    g = jnp.dot(xnb, gbf[:], preferred_element_type=jnp.float32)
    u = jnp.dot(xnb, ubf[:], preferred_element_type=jnp.float32)
    h = (g * jax.nn.sigmoid(g)) * u
    o = jnp.dot(h.astype(jnp.bfloat16), dbf[:], preferred_element_type=jnp.float32)
    iota = jax.lax.broadcasted_iota(jnp.int32, (_TB, 128), 1)
    w = jnp.sum(jnp.where(iota == e, comb_ref[pl.ds(ts, _TB), :], 0.0),
                axis=1, keepdims=True)
    contrib = o * w

    @pl.when(jnp.logical_and(e == 0, f == 0))
    def _init():
        out_ref[pl.ds(ts, _TB), :] = x + contrib

    @pl.when(jnp.logical_or(e != 0, f != 0))
    def _acc():
        out_ref[pl.ds(ts, _TB), :] = out_ref[pl.ds(ts, _TB), :] + contrib


def kernel(hidden_states, rms_weight, router_weight, router_bias,
           gate_proj, up_proj, down_proj):
    b, s, d = hidden_states.shape
    e_num, _, f_dim = gate_proj.shape
    t_tok = b * s
    nt = t_tok // _TB
    nf = f_dim // _FB
    xt = hidden_states.reshape(t_tok, d)
    rw_pad = jnp.zeros((d, 128), jnp.float32).at[:, :e_num].set(router_weight.T)
    rb_pad = jnp.full((1, 128), -1e30, jnp.float32).at[0, :e_num].set(router_bias)
    rms2 = rms_weight.reshape(1, d)

    out = pl.pallas_call(
        _moe_body,
        grid=(e_num, nf, nt),
        in_specs=[
            pl.BlockSpec((t_tok, d), lambda e, f, t: (0, 0)),
            pl.BlockSpec((1, d), lambda e, f, t: (0, 0)),
            pl.BlockSpec((d, 128), lambda e, f, t: (0, 0)),
            pl.BlockSpec((1, 128), lambda e, f, t: (0, 0)),
            pl.BlockSpec((1, d, _FB), lambda e, f, t: (e, 0, f)),
            pl.BlockSpec((1, d, _FB), lambda e, f, t: (e, 0, f)),
            pl.BlockSpec((1, _FB, d), lambda e, f, t: (e, f, 0)),
        ],
        out_specs=pl.BlockSpec((t_tok, d), lambda e, f, t: (0, 0)),
        out_shape=jax.ShapeDtypeStruct((t_tok, d), jnp.float32),
        scratch_shapes=[
            pltpu.VMEM((t_tok, 128), jnp.float32),
            pltpu.VMEM((d, _FB), jnp.bfloat16),
            pltpu.VMEM((d, _FB), jnp.bfloat16),
            pltpu.VMEM((_FB, d), jnp.bfloat16),
        ],
        compiler_params=pltpu.CompilerParams(
            dimension_semantics=("arbitrary", "arbitrary", "arbitrary")),
    )(xt, rms2, rw_pad, rb_pad, gate_proj, up_proj, down_proj)
    return out.reshape(b, s, d)


# speedup vs baseline: 1.0877x; 1.0877x over previous
import jax
import jax.numpy as jnp
from jax.experimental import pallas as pl
from jax.experimental.pallas import tpu as pltpu

_EPS = 1e-6
_TB = 256
_FB = 256


def _router_body(xt_ref, rms_ref, rw_ref, rb_ref, xn_ref, comb_ref):
    x = xt_ref[:]
    var = jnp.mean(x * x, axis=1, keepdims=True)
    xn = (x * jax.lax.rsqrt(var + _EPS)) * rms_ref[:]
    xn_ref[:] = xn.astype(jnp.bfloat16)
    logits = jnp.dot(xn, rw_ref[:], preferred_element_type=jnp.float32)
    scores = jax.nn.sigmoid(logits)
    choice = scores + rb_ref[:]
    iota = jax.lax.broadcasted_iota(jnp.int32, (_TB, 128), 1)
    m1 = jnp.max(choice, axis=1, keepdims=True)
    i1 = jnp.min(jnp.where(choice == m1, iota, 128), axis=1, keepdims=True)
    mask1 = iota == i1
    choice2 = jnp.where(mask1, -jnp.inf, choice)
    m2 = jnp.max(choice2, axis=1, keepdims=True)
    i2 = jnp.min(jnp.where(choice2 == m2, iota, 128), axis=1, keepdims=True)
    mask2 = iota == i2
    a1 = jnp.sum(jnp.where(mask1, scores, 0.0), axis=1, keepdims=True)
    a2 = jnp.sum(jnp.where(mask2, scores, 0.0), axis=1, keepdims=True)
    den = a1 + a2 + 1e-9
    comb_ref[:] = (jnp.where(mask1, a1 / den, 0.0)
                   + jnp.where(mask2, a2 / den, 0.0))


def _moe_body(xn_ref, comb_ref, xt_ref, g_ref, u_ref, d_ref,
              out_ref, gbf, ubf, dbf, resid, sem):
    e = pl.program_id(0)
    f = pl.program_id(1)
    t = pl.program_id(2)
    ts = t * _TB

    @pl.when(t == 0)
    def _cast_weights():
        gbf[:] = g_ref[0].astype(jnp.bfloat16)
        ubf[:] = u_ref[0].astype(jnp.bfloat16)
        dbf[:] = d_ref[0].astype(jnp.bfloat16)

    xnb = xn_ref[pl.ds(ts, _TB), :]
    g = jnp.dot(xnb, gbf[:], preferred_element_type=jnp.float32)
    u = jnp.dot(xnb, ubf[:], preferred_element_type=jnp.float32)
    h = (g * jax.nn.sigmoid(g)) * u
    o = jnp.dot(h.astype(jnp.bfloat16), dbf[:], preferred_element_type=jnp.float32)
    onehot = (jax.lax.broadcasted_iota(jnp.int32, (128, 1), 0) == e
              ).astype(jnp.float32)
    w = jnp.dot(comb_ref[pl.ds(ts, _TB), :], onehot,
                preferred_element_type=jnp.float32)
    contrib = o * w

    @pl.when(jnp.logical_and(e == 0, f == 0))
    def _init():
        pltpu.make_async_copy(
            xt_ref.at[pl.ds(ts, _TB), :], resid, sem).start()
        pltpu.make_async_copy(
            xt_ref.at[pl.ds(ts, _TB), :], resid, sem).wait()
        out_ref[pl.ds(ts, _TB), :] = resid[:] + contrib

    @pl.when(jnp.logical_or(e != 0, f != 0))
    def _acc():
        out_ref[pl.ds(ts, _TB), :] = out_ref[pl.ds(ts, _TB), :] + contrib


def kernel(hidden_states, rms_weight, router_weight, router_bias,
           gate_proj, up_proj, down_proj):
    b, s, d = hidden_states.shape
    e_num, _, f_dim = gate_proj.shape
    t_tok = b * s
    nt = t_tok // _TB
    nf = f_dim // _FB
    xt = hidden_states.reshape(t_tok, d)
    rw_pad = jnp.zeros((d, 128), jnp.float32).at[:, :e_num].set(router_weight.T)
    rb_pad = jnp.full((1, 128), -1e30, jnp.float32).at[0, :e_num].set(router_bias)
    rms2 = rms_weight.reshape(1, d)

    xn, comb = pl.pallas_call(
        _router_body,
        grid=(nt,),
        in_specs=[
            pl.BlockSpec((_TB, d), lambda t: (t, 0)),
            pl.BlockSpec((1, d), lambda t: (0, 0)),
            pl.BlockSpec((d, 128), lambda t: (0, 0)),
            pl.BlockSpec((1, 128), lambda t: (0, 0)),
        ],
        out_specs=[
            pl.BlockSpec((_TB, d), lambda t: (t, 0)),
            pl.BlockSpec((_TB, 128), lambda t: (t, 0)),
        ],
        out_shape=[
            jax.ShapeDtypeStruct((t_tok, d), jnp.bfloat16),
            jax.ShapeDtypeStruct((t_tok, 128), jnp.float32),
        ],
    )(xt, rms2, rw_pad, rb_pad)

    out = pl.pallas_call(
        _moe_body,
        grid=(e_num, nf, nt),
        in_specs=[
            pl.BlockSpec((t_tok, d), lambda e, f, t: (0, 0)),
            pl.BlockSpec((t_tok, 128), lambda e, f, t: (0, 0)),
            pl.BlockSpec(memory_space=pl.ANY),
            pl.BlockSpec((1, d, _FB), lambda e, f, t: (e, 0, f)),
            pl.BlockSpec((1, d, _FB), lambda e, f, t: (e, 0, f)),
            pl.BlockSpec((1, _FB, d), lambda e, f, t: (e, f, 0)),
        ],
        out_specs=pl.BlockSpec((t_tok, d), lambda e, f, t: (0, 0)),
        out_shape=jax.ShapeDtypeStruct((t_tok, d), jnp.float32),
        scratch_shapes=[
            pltpu.VMEM((d, _FB), jnp.bfloat16),
            pltpu.VMEM((d, _FB), jnp.bfloat16),
            pltpu.VMEM((_FB, d), jnp.bfloat16),
            pltpu.VMEM((_TB, d), jnp.float32),
            pltpu.SemaphoreType.DMA,
        ],
        compiler_params=pltpu.CompilerParams(
            dimension_semantics=("arbitrary", "arbitrary", "arbitrary")),
    )(xn, comb, xt, gate_proj, up_proj, down_proj)
    return out.reshape(b, s, d)
